# baseline (speedup 1.0000x reference)
import functools

import jax
import jax.numpy as jnp
from jax import lax
from jax.experimental import pallas as pl
from jax.experimental.pallas import tpu as pltpu
import jax.experimental.pallas.tpu_sc as plsc

N = 10000
E = 320000
NC, NS, L = 2, 16, 16
NW = NC * NS
NPAD = 10240
R = 400
CH = 128
NBUF = 2
G = 1
CH3 = 128
NBUF3 = 4
G3 = 2


def _sc_degree(dst):
    epw = E // NW
    cpt = NPAD // NS
    mesh = plsc.VectorSubcoreMesh(core_axis_name="c", subcore_axis_name="s")

    @functools.partial(
        pl.kernel,
        out_type=jax.ShapeDtypeStruct((2 * NPAD,), jnp.float32),
        mesh=mesh,
        scratch_types=[
            pltpu.VMEM((epw,), jnp.int32),
            pltpu.VMEM((NPAD,), jnp.float32),
            pltpu.VMEM((NS * cpt,), jnp.float32),
            pltpu.VMEM((cpt,), jnp.float32),
            pltpu.VMEM_SHARED((NS * NPAD,), jnp.float32),
        ],
        compiler_params=pltpu.CompilerParams(needs_layout_passes=False),
    )
    def k(dst_hbm, out_hbm, dst_v, hist_v, red_v, res_v, shared):
        c = lax.axis_index("c")
        s = lax.axis_index("s")
        wid = s * NC + c

        pltpu.sync_copy(dst_hbm.at[pl.ds(wid * epw, epw)], dst_v)

        def zero_body(j, _):
            hist_v[pl.ds(j * L, L)] = jnp.zeros((L,), jnp.float32)
            return 0
        lax.fori_loop(0, NPAD // L, zero_body, 0)

        ones = jnp.full((L,), 1.0, jnp.float32)

        def count_body(j, _):
            idx = dst_v[pl.ds(j * L, L)]
            plsc.addupdate_scatter(hist_v, [idx], ones)
            return 0
        lax.fori_loop(0, epw // L, count_body, 0)

        pltpu.sync_copy(hist_v, shared.at[pl.ds(s * NPAD, NPAD)])
        plsc.subcore_barrier()

        for t in range(NS):
            pltpu.sync_copy(shared.at[pl.ds(t * NPAD + s * cpt, cpt)],
                            red_v.at[pl.ds(t * cpt, cpt)])

        def red_body(j, _):
            acc = jnp.zeros((L,), jnp.float32)

            def add_t(t, a):
                return a + red_v[pl.ds(t * cpt + j * L, L)]
            acc = lax.fori_loop(0, NS, add_t, acc)
            res_v[pl.ds(j * L, L)] = acc
            return 0
        lax.fori_loop(0, cpt // L, red_body, 0)

        pltpu.sync_copy(res_v, out_hbm.at[pl.ds(c * NPAD + s * cpt, cpt)])

    return k(dst)


def _sc_propagate(y0, y1, ecnk, fh, ch, nbuf, g):
    nch = ecnk.shape[0] // NS
    NI = 2 * nbuf
    ngrp2 = nch // NI
    tail = nch - ngrp2 * NI
    rpt = NPAD // NS
    mesh = plsc.VectorSubcoreMesh(core_axis_name="c", subcore_axis_name="s")
    half = jax.ShapeDtypeStruct((NPAD, fh), jnp.float32)

    @functools.partial(
        pl.kernel,
        out_type=(half, half),
        mesh=mesh,
        scratch_types=[
            pltpu.VMEM((NI, 2, ch), jnp.int32),
            pltpu.VMEM((nbuf, ch, fh), jnp.float32),
            pltpu.VMEM_SHARED((NPAD, fh), jnp.float32),
            [pltpu.SemaphoreType.DMA] * NI,
            [pltpu.SemaphoreType.DMA] * nbuf,
            [pltpu.SemaphoreType.DMA] * nbuf,
        ],
        compiler_params=pltpu.CompilerParams(needs_layout_passes=False,
                                             use_tc_tiling_on_sc=False),
    )
    def k(y0_hbm, y1_hbm, e_hbm, z0_hbm, z1_hbm,
          ibuf, gbuf, accum, isems, gsems, ssems):
        c = lax.axis_index("c")
        s = lax.axis_index("s")
        base = s * nch

        rlo = s * rpt

        @pl.when(c == 0)
        def _():
            pltpu.sync_copy(y0_hbm.at[pl.ds(rlo, rpt)],
                            accum.at[pl.ds(rlo, rpt)])

        @pl.when(c == 1)
        def _():
            pltpu.sync_copy(y1_hbm.at[pl.ds(rlo, rpt)],
                            accum.at[pl.ds(rlo, rpt)])

        def fire_idx(j, islot):
            pltpu.async_copy(e_hbm.at[base + j], ibuf.at[islot], isems[islot])

        def wait_idx(islot):
            pltpu.make_async_copy(e_hbm.at[0], ibuf.at[islot],
                                  isems[islot]).wait()

        def fire_gather(islot, slot):
            idx = ibuf.at[islot, 0]

            @pl.when(c == 0)
            def _():
                pltpu.async_copy(y0_hbm.at[idx], gbuf.at[slot], gsems[slot])

            @pl.when(c == 1)
            def _():
                pltpu.async_copy(y1_hbm.at[idx], gbuf.at[slot], gsems[slot])

        def wait_gather(slot):
            pltpu.make_async_copy(y0_hbm.at[pl.ds(0, ch)], gbuf.at[slot],
                                  gsems[slot]).wait()

        def fire_scatter(islot, slot):
            pltpu.async_copy(gbuf.at[slot], accum.at[ibuf.at[islot, 1]],
                             ssems[slot], add=True)

        def drain_scatter(slot):
            pltpu.make_async_copy(gbuf.at[slot], accum.at[ibuf.at[0, 1]],
                                  ssems[slot]).wait()

        plsc.subcore_barrier()

        for b in range(nbuf):
            fire_idx(b, b)
        for b in range(g):
            wait_idx(b)
            fire_gather(b, b)

        def chunk_step(j, b, islot):
            gslot = (b + g) % nbuf
            gislot = (islot + g) % NI

            @pl.when(j < nch - g)
            def _():
                @pl.when(j + g >= nbuf)
                def _():
                    drain_scatter(gslot)
                wait_idx(gislot)
                fire_gather(gislot, gslot)

            wait_gather(b)
            fire_scatter(islot, b)

            @pl.when(j < nch - nbuf)
            def _():
                fire_idx(j + nbuf, (islot + nbuf) % NI)

        def outer(g2, _):
            for p in range(NI):
                j = g2 * NI + p
                chunk_step(j, p % nbuf, p)
            return 0
        lax.fori_loop(0, ngrp2, outer, 0)

        for p in range(tail):
            j = ngrp2 * NI + p
            chunk_step(jnp.int32(j), p % nbuf, p)

        for b in range(nbuf):
            drain_scatter(b)

        plsc.subcore_barrier()

        @pl.when(c == 0)
        def _():
            pltpu.sync_copy(accum.at[pl.ds(rlo, rpt)],
                            z0_hbm.at[pl.ds(rlo, rpt)])

        @pl.when(c == 1)
        def _():
            pltpu.sync_copy(accum.at[pl.ds(rlo, rpt)],
                            z1_hbm.at[pl.ds(rlo, rpt)])

    return k(y0, y1, ecnk)


def _full(shape):
    nd = len(shape)
    return pl.BlockSpec(shape, lambda i: (0,) * nd)


def _row(block):
    return pl.BlockSpec(block, lambda i: (i,) + (0,) * (len(block) - 1))


def _halves_out(fo):
    h = jax.ShapeDtypeStruct((NPAD, fo // 2), jnp.float32)
    return dict(
        out_shape=(h, h),
        out_specs=(_row((R, fo // 2)), _row((R, fo // 2))),
    )


def _tc_layer1(x, W, deg2):
    fo = W.shape[1]

    def body(x_ref, w_ref, d_ref, o0_ref, o1_ref):
        dinv = lax.rsqrt(d_ref[...] + 1.0)
        y = jnp.dot(x_ref[...], w_ref[...],
                    preferred_element_type=jnp.float32) * dinv
        o0_ref[...] = y[:, : fo // 2]
        o1_ref[...] = y[:, fo // 2:]

    return pl.pallas_call(
        body,
        grid=(N // R,),
        in_specs=[_row((R, x.shape[1])), _full(W.shape), _row((R, 1))],
        **_halves_out(fo),
    )(x, W, deg2)


def _tc_layer(z0, z1, b, W, deg2):
    fo = W.shape[1]
    fh = z0.shape[1]

    def body(z0_ref, z1_ref, b_ref, w_ref, d_ref, o0_ref, o1_ref):
        dinv = lax.rsqrt(d_ref[...] + 1.0)
        zc = jnp.concatenate([z0_ref[...], z1_ref[...]], axis=1)
        h = jnp.maximum(zc * dinv + b_ref[...], 0.0)
        y = jnp.dot(h, w_ref[...], preferred_element_type=jnp.float32) * dinv
        o0_ref[...] = y[:, : fo // 2]
        o1_ref[...] = y[:, fo // 2:]

    return pl.pallas_call(
        body,
        grid=(N // R,),
        in_specs=[_row((R, fh)), _row((R, fh)), _full(b.shape),
                  _full(W.shape), _row((R, 1))],
        **_halves_out(fo),
    )(z0, z1, b, W, deg2)


def _tc_final(z0, z1, b, deg2):
    fh = z0.shape[1]

    def body(z0_ref, z1_ref, b_ref, d_ref, o_ref):
        dinv = lax.rsqrt(d_ref[...] + 1.0)
        zc = jnp.concatenate([z0_ref[...], z1_ref[...]], axis=1)
        o_ref[...] = jax.nn.sigmoid(zc * dinv + b_ref[...])

    return pl.pallas_call(
        body,
        grid=(N // R,),
        in_specs=[_row((R, fh)), _row((R, fh)), _full(b.shape), _row((R, 1))],
        out_shape=jax.ShapeDtypeStruct((N, 2 * fh), jnp.float32),
        out_specs=_row((R, 2 * fh)),
    )(z0, z1, b, deg2)


def _chunk_edges(src, dst, ch):
    ept = E // NS
    cpt = -(-ept // ch)
    npe = cpt * ch - ept
    s16 = src.reshape(NS, ept)
    d16 = dst.reshape(NS, ept)
    if npe:
        flat = jnp.arange(NS * npe, dtype=jnp.int32).reshape(NS, npe)
        s16 = jnp.concatenate([s16, flat % N], axis=1)
        d16 = jnp.concatenate([d16, N + flat % (NPAD - N)], axis=1)
    s16 = s16.reshape(NS, cpt, ch)
    d16 = d16.reshape(NS, cpt, ch)
    return jnp.stack([s16, d16], axis=2).reshape(NS * cpt, 2, ch)


def kernel(x, edge_index, W1, b1, W2, b2, W3, b3):
    src = edge_index[0].astype(jnp.int32)
    dst = edge_index[1].astype(jnp.int32)
    ecnk = _chunk_edges(src, dst, CH)
    ecnk3 = _chunk_edges(src, dst, CH3)

    degf = _sc_degree(dst)
    deg2 = (degf[:N] + degf[NPAD:NPAD + N]).reshape(N, 1)

    y0, y1 = _tc_layer1(x, W1, deg2)
    z0, z1 = _sc_propagate(y0, y1, ecnk, 128, CH, NBUF, G)

    y0, y1 = _tc_layer(z0, z1, b1.reshape(1, -1), W2, deg2)
    z0, z1 = _sc_propagate(y0, y1, ecnk, 128, CH, NBUF, G)

    y0, y1 = _tc_layer(z0, z1, b2.reshape(1, -1), W3, deg2)
    z0, z1 = _sc_propagate(y0, y1, ecnk3, 32, CH3, NBUF3, G3)

    return _tc_final(z0, z1, b3.reshape(1, -1), deg2)


# speedup vs baseline: 1.0297x; 1.0297x over previous
import functools

import jax
import jax.numpy as jnp
from jax import lax
from jax.experimental import pallas as pl
from jax.experimental.pallas import tpu as pltpu
import jax.experimental.pallas.tpu_sc as plsc

N = 10000
E = 320000
NC, NS, L = 2, 16, 16
NW = NC * NS
NPAD = 10240
R = 400
CH = 88
NBUF = 4
G = 2
CH3 = 128
NBUF3 = 4
G3 = 2


def _sc_degree(dst):
    epw = E // NW
    cpt = NPAD // NS
    mesh = plsc.VectorSubcoreMesh(core_axis_name="c", subcore_axis_name="s")

    @functools.partial(
        pl.kernel,
        out_type=jax.ShapeDtypeStruct((2 * NPAD,), jnp.float32),
        mesh=mesh,
        scratch_types=[
            pltpu.VMEM((epw,), jnp.int32),
            pltpu.VMEM((NPAD,), jnp.float32),
            pltpu.VMEM((NS * cpt,), jnp.float32),
            pltpu.VMEM((cpt,), jnp.float32),
            pltpu.VMEM_SHARED((NS * NPAD,), jnp.float32),
        ],
        compiler_params=pltpu.CompilerParams(needs_layout_passes=False),
    )
    def k(dst_hbm, out_hbm, dst_v, hist_v, red_v, res_v, shared):
        c = lax.axis_index("c")
        s = lax.axis_index("s")
        wid = s * NC + c

        pltpu.sync_copy(dst_hbm.at[pl.ds(wid * epw, epw)], dst_v)

        def zero_body(j, _):
            hist_v[pl.ds(j * L, L)] = jnp.zeros((L,), jnp.float32)
            return 0
        lax.fori_loop(0, NPAD // L, zero_body, 0)

        ones = jnp.full((L,), 1.0, jnp.float32)

        def count_body(j, _):
            idx = dst_v[pl.ds(j * L, L)]
            plsc.addupdate_scatter(hist_v, [idx], ones)
            return 0
        lax.fori_loop(0, epw // L, count_body, 0)

        pltpu.sync_copy(hist_v, shared.at[pl.ds(s * NPAD, NPAD)])
        plsc.subcore_barrier()

        for t in range(NS):
            pltpu.sync_copy(shared.at[pl.ds(t * NPAD + s * cpt, cpt)],
                            red_v.at[pl.ds(t * cpt, cpt)])

        def red_body(j, _):
            acc = jnp.zeros((L,), jnp.float32)

            def add_t(t, a):
                return a + red_v[pl.ds(t * cpt + j * L, L)]
            acc = lax.fori_loop(0, NS, add_t, acc)
            res_v[pl.ds(j * L, L)] = acc
            return 0
        lax.fori_loop(0, cpt // L, red_body, 0)

        pltpu.sync_copy(res_v, out_hbm.at[pl.ds(c * NPAD + s * cpt, cpt)])

    return k(dst)


def _sc_propagate(y0, y1, ecnk, fh, ch, nbuf, g):
    nch = ecnk.shape[0] // NS
    NI = 2 * nbuf
    ngrp2 = nch // NI
    tail = nch - ngrp2 * NI
    rpt = NPAD // NS
    mesh = plsc.VectorSubcoreMesh(core_axis_name="c", subcore_axis_name="s")
    half = jax.ShapeDtypeStruct((NPAD, fh), jnp.float32)

    @functools.partial(
        pl.kernel,
        out_type=(half, half),
        mesh=mesh,
        scratch_types=[
            pltpu.VMEM((NI, 2, ch), jnp.int32),
            pltpu.VMEM((nbuf, ch, fh), jnp.float32),
            pltpu.VMEM_SHARED((NPAD, fh), jnp.float32),
            [pltpu.SemaphoreType.DMA] * NI,
            [pltpu.SemaphoreType.DMA] * nbuf,
            [pltpu.SemaphoreType.DMA] * nbuf,
        ],
        compiler_params=pltpu.CompilerParams(needs_layout_passes=False,
                                             use_tc_tiling_on_sc=False),
    )
    def k(y0_hbm, y1_hbm, e_hbm, z0_hbm, z1_hbm,
          ibuf, gbuf, accum, isems, gsems, ssems):
        c = lax.axis_index("c")
        s = lax.axis_index("s")
        base = s * nch

        rlo = s * rpt

        @pl.when(c == 0)
        def _():
            pltpu.sync_copy(y0_hbm.at[pl.ds(rlo, rpt)],
                            accum.at[pl.ds(rlo, rpt)])

        @pl.when(c == 1)
        def _():
            pltpu.sync_copy(y1_hbm.at[pl.ds(rlo, rpt)],
                            accum.at[pl.ds(rlo, rpt)])

        def fire_idx(j, islot):
            pltpu.async_copy(e_hbm.at[base + j], ibuf.at[islot], isems[islot])

        def wait_idx(islot):
            pltpu.make_async_copy(e_hbm.at[0], ibuf.at[islot],
                                  isems[islot]).wait()

        def fire_gather(islot, slot):
            idx = ibuf.at[islot, 0]

            @pl.when(c == 0)
            def _():
                pltpu.async_copy(y0_hbm.at[idx], gbuf.at[slot], gsems[slot])

            @pl.when(c == 1)
            def _():
                pltpu.async_copy(y1_hbm.at[idx], gbuf.at[slot], gsems[slot])

        def wait_gather(slot):
            pltpu.make_async_copy(y0_hbm.at[pl.ds(0, ch)], gbuf.at[slot],
                                  gsems[slot]).wait()

        def fire_scatter(islot, slot):
            pltpu.async_copy(gbuf.at[slot], accum.at[ibuf.at[islot, 1]],
                             ssems[slot], add=True)

        def drain_scatter(slot):
            pltpu.make_async_copy(gbuf.at[slot], accum.at[ibuf.at[0, 1]],
                                  ssems[slot]).wait()

        plsc.subcore_barrier()

        for b in range(nbuf):
            fire_idx(b, b)
        for b in range(g):
            wait_idx(b)
            fire_gather(b, b)

        def chunk_step(j, b, islot):
            gslot = (b + g) % nbuf
            gislot = (islot + g) % NI

            @pl.when(j < nch - g)
            def _():
                @pl.when(j + g >= nbuf)
                def _():
                    drain_scatter(gslot)
                wait_idx(gislot)
                fire_gather(gislot, gslot)

            wait_gather(b)
            fire_scatter(islot, b)

            @pl.when(j < nch - nbuf)
            def _():
                fire_idx(j + nbuf, (islot + nbuf) % NI)

        def outer(g2, _):
            for p in range(NI):
                j = g2 * NI + p
                chunk_step(j, p % nbuf, p)
            return 0
        lax.fori_loop(0, ngrp2, outer, 0)

        for p in range(tail):
            j = ngrp2 * NI + p
            chunk_step(jnp.int32(j), p % nbuf, p)

        for b in range(nbuf):
            drain_scatter(b)

        plsc.subcore_barrier()

        @pl.when(c == 0)
        def _():
            pltpu.sync_copy(accum.at[pl.ds(rlo, rpt)],
                            z0_hbm.at[pl.ds(rlo, rpt)])

        @pl.when(c == 1)
        def _():
            pltpu.sync_copy(accum.at[pl.ds(rlo, rpt)],
                            z1_hbm.at[pl.ds(rlo, rpt)])

    return k(y0, y1, ecnk)


def _full(shape):
    nd = len(shape)
    return pl.BlockSpec(shape, lambda i: (0,) * nd)


def _row(block):
    return pl.BlockSpec(block, lambda i: (i,) + (0,) * (len(block) - 1))


def _halves_out(fo):
    h = jax.ShapeDtypeStruct((NPAD, fo // 2), jnp.float32)
    return dict(
        out_shape=(h, h),
        out_specs=(_row((R, fo // 2)), _row((R, fo // 2))),
    )


def _tc_layer1(x, W, deg2):
    fo = W.shape[1]

    def body(x_ref, w_ref, d_ref, o0_ref, o1_ref):
        dinv = lax.rsqrt(d_ref[...] + 1.0)
        y = jnp.dot(x_ref[...], w_ref[...],
                    preferred_element_type=jnp.float32) * dinv
        o0_ref[...] = y[:, : fo // 2]
        o1_ref[...] = y[:, fo // 2:]

    return pl.pallas_call(
        body,
        grid=(N // R,),
        in_specs=[_row((R, x.shape[1])), _full(W.shape), _row((R, 1))],
        **_halves_out(fo),
    )(x, W, deg2)


def _tc_layer(z0, z1, b, W, deg2):
    fo = W.shape[1]
    fh = z0.shape[1]

    def body(z0_ref, z1_ref, b_ref, w_ref, d_ref, o0_ref, o1_ref):
        dinv = lax.rsqrt(d_ref[...] + 1.0)
        zc = jnp.concatenate([z0_ref[...], z1_ref[...]], axis=1)
        h = jnp.maximum(zc * dinv + b_ref[...], 0.0)
        y = jnp.dot(h, w_ref[...], preferred_element_type=jnp.float32) * dinv
        o0_ref[...] = y[:, : fo // 2]
        o1_ref[...] = y[:, fo // 2:]

    return pl.pallas_call(
        body,
        grid=(N // R,),
        in_specs=[_row((R, fh)), _row((R, fh)), _full(b.shape),
                  _full(W.shape), _row((R, 1))],
        **_halves_out(fo),
    )(z0, z1, b, W, deg2)


def _tc_final(z0, z1, b, deg2):
    fh = z0.shape[1]

    def body(z0_ref, z1_ref, b_ref, d_ref, o_ref):
        dinv = lax.rsqrt(d_ref[...] + 1.0)
        zc = jnp.concatenate([z0_ref[...], z1_ref[...]], axis=1)
        o_ref[...] = jax.nn.sigmoid(zc * dinv + b_ref[...])

    return pl.pallas_call(
        body,
        grid=(N // R,),
        in_specs=[_row((R, fh)), _row((R, fh)), _full(b.shape), _row((R, 1))],
        out_shape=jax.ShapeDtypeStruct((N, 2 * fh), jnp.float32),
        out_specs=_row((R, 2 * fh)),
    )(z0, z1, b, deg2)


def _chunk_edges(src, dst, ch):
    ept = E // NS
    cpt = -(-ept // ch)
    npe = cpt * ch - ept
    s16 = src.reshape(NS, ept)
    d16 = dst.reshape(NS, ept)
    if npe:
        flat = jnp.arange(NS * npe, dtype=jnp.int32).reshape(NS, npe)
        s16 = jnp.concatenate([s16, flat % N], axis=1)
        d16 = jnp.concatenate([d16, N + flat % (NPAD - N)], axis=1)
    s16 = s16.reshape(NS, cpt, ch)
    d16 = d16.reshape(NS, cpt, ch)
    return jnp.stack([s16, d16], axis=2).reshape(NS * cpt, 2, ch)


def kernel(x, edge_index, W1, b1, W2, b2, W3, b3):
    src = edge_index[0].astype(jnp.int32)
    dst = edge_index[1].astype(jnp.int32)
    ecnk = _chunk_edges(src, dst, CH)
    ecnk3 = _chunk_edges(src, dst, CH3)

    degf = _sc_degree(dst)
    deg2 = (degf[:N] + degf[NPAD:NPAD + N]).reshape(N, 1)

    y0, y1 = _tc_layer1(x, W1, deg2)
    z0, z1 = _sc_propagate(y0, y1, ecnk, 128, CH, NBUF, G)

    y0, y1 = _tc_layer(z0, z1, b1.reshape(1, -1), W2, deg2)
    z0, z1 = _sc_propagate(y0, y1, ecnk, 128, CH, NBUF, G)

    y0, y1 = _tc_layer(z0, z1, b2.reshape(1, -1), W3, deg2)
    z0, z1 = _sc_propagate(y0, y1, ecnk3, 32, CH3, NBUF3, G3)

    return _tc_final(z0, z1, b3.reshape(1, -1), deg2)


# speedup vs baseline: 1.0582x; 1.0277x over previous
import functools

import jax
import jax.numpy as jnp
from jax import lax
from jax.experimental import pallas as pl
from jax.experimental.pallas import tpu as pltpu
import jax.experimental.pallas.tpu_sc as plsc

N = 10000
E = 320000
NC, NS, L = 2, 16, 16
NW = NC * NS
NPAD = 10240
R = 400
CH = 96
NBUF = 3
G = 2
CH3 = 128
NBUF3 = 4
G3 = 2


def _sc_degree(dst):
    epw = E // NW
    cpt = NPAD // NS
    mesh = plsc.VectorSubcoreMesh(core_axis_name="c", subcore_axis_name="s")

    @functools.partial(
        pl.kernel,
        out_type=jax.ShapeDtypeStruct((2 * NPAD,), jnp.float32),
        mesh=mesh,
        scratch_types=[
            pltpu.VMEM((epw,), jnp.int32),
            pltpu.VMEM((NPAD,), jnp.float32),
            pltpu.VMEM((NS * cpt,), jnp.float32),
            pltpu.VMEM((cpt,), jnp.float32),
            pltpu.VMEM_SHARED((NS * NPAD,), jnp.float32),
        ],
        compiler_params=pltpu.CompilerParams(needs_layout_passes=False),
    )
    def k(dst_hbm, out_hbm, dst_v, hist_v, red_v, res_v, shared):
        c = lax.axis_index("c")
        s = lax.axis_index("s")
        wid = s * NC + c

        pltpu.sync_copy(dst_hbm.at[pl.ds(wid * epw, epw)], dst_v)

        def zero_body(j, _):
            hist_v[pl.ds(j * L, L)] = jnp.zeros((L,), jnp.float32)
            return 0
        lax.fori_loop(0, NPAD // L, zero_body, 0)

        ones = jnp.full((L,), 1.0, jnp.float32)

        def count_body(j, _):
            idx = dst_v[pl.ds(j * L, L)]
            plsc.addupdate_scatter(hist_v, [idx], ones)
            return 0
        lax.fori_loop(0, epw // L, count_body, 0)

        pltpu.sync_copy(hist_v, shared.at[pl.ds(s * NPAD, NPAD)])
        plsc.subcore_barrier()

        for t in range(NS):
            pltpu.sync_copy(shared.at[pl.ds(t * NPAD + s * cpt, cpt)],
                            red_v.at[pl.ds(t * cpt, cpt)])

        def red_body(j, _):
            acc = jnp.zeros((L,), jnp.float32)

            def add_t(t, a):
                return a + red_v[pl.ds(t * cpt + j * L, L)]
            acc = lax.fori_loop(0, NS, add_t, acc)
            res_v[pl.ds(j * L, L)] = acc
            return 0
        lax.fori_loop(0, cpt // L, red_body, 0)

        pltpu.sync_copy(res_v, out_hbm.at[pl.ds(c * NPAD + s * cpt, cpt)])

    return k(dst)


def _sc_propagate(y0, y1, ecnk, fh, ch, nbuf, g):
    nch = ecnk.shape[0] // NS
    NI = 2 * nbuf
    ngrp2 = nch // NI
    tail = nch - ngrp2 * NI
    rpt = NPAD // NS
    mesh = plsc.VectorSubcoreMesh(core_axis_name="c", subcore_axis_name="s")
    half = jax.ShapeDtypeStruct((NPAD, fh), jnp.float32)

    @functools.partial(
        pl.kernel,
        out_type=(half, half),
        mesh=mesh,
        scratch_types=[
            pltpu.VMEM((NI, 2, ch), jnp.int32),
            pltpu.VMEM((nbuf, ch, fh), jnp.float32),
            pltpu.VMEM_SHARED((NPAD, fh), jnp.float32),
            [pltpu.SemaphoreType.DMA] * NI,
            [pltpu.SemaphoreType.DMA] * nbuf,
            [pltpu.SemaphoreType.DMA] * nbuf,
        ],
        compiler_params=pltpu.CompilerParams(needs_layout_passes=False,
                                             use_tc_tiling_on_sc=False),
    )
    def k(y0_hbm, y1_hbm, e_hbm, z0_hbm, z1_hbm,
          ibuf, gbuf, accum, isems, gsems, ssems):
        c = lax.axis_index("c")
        s = lax.axis_index("s")
        base = s * nch

        rlo = s * rpt

        @pl.when(c == 0)
        def _():
            pltpu.sync_copy(y0_hbm.at[pl.ds(rlo, rpt)],
                            accum.at[pl.ds(rlo, rpt)])

        @pl.when(c == 1)
        def _():
            pltpu.sync_copy(y1_hbm.at[pl.ds(rlo, rpt)],
                            accum.at[pl.ds(rlo, rpt)])

        def fire_idx(j, islot):
            pltpu.async_copy(e_hbm.at[base + j], ibuf.at[islot], isems[islot])

        def wait_idx(islot):
            pltpu.make_async_copy(e_hbm.at[0], ibuf.at[islot],
                                  isems[islot]).wait()

        def fire_gather(islot, slot):
            idx = ibuf.at[islot, 0]

            @pl.when(c == 0)
            def _():
                pltpu.async_copy(y0_hbm.at[idx], gbuf.at[slot], gsems[slot])

            @pl.when(c == 1)
            def _():
                pltpu.async_copy(y1_hbm.at[idx], gbuf.at[slot], gsems[slot])

        def wait_gather(slot):
            pltpu.make_async_copy(y0_hbm.at[pl.ds(0, ch)], gbuf.at[slot],
                                  gsems[slot]).wait()

        def fire_scatter(islot, slot):
            pltpu.async_copy(gbuf.at[slot], accum.at[ibuf.at[islot, 1]],
                             ssems[slot], add=True)

        def drain_scatter(slot):
            pltpu.make_async_copy(gbuf.at[slot], accum.at[ibuf.at[0, 1]],
                                  ssems[slot]).wait()

        plsc.subcore_barrier()

        for b in range(nbuf):
            fire_idx(b, b)
        for b in range(g):
            wait_idx(b)
            fire_gather(b, b)

        def chunk_step(j, b, islot):
            gslot = (b + g) % nbuf
            gislot = (islot + g) % NI

            @pl.when(j < nch - g)
            def _():
                @pl.when(j + g >= nbuf)
                def _():
                    drain_scatter(gslot)
                wait_idx(gislot)
                fire_gather(gislot, gslot)

            wait_gather(b)
            fire_scatter(islot, b)

            @pl.when(j < nch - nbuf)
            def _():
                fire_idx(j + nbuf, (islot + nbuf) % NI)

        def outer(g2, _):
            for p in range(NI):
                j = g2 * NI + p
                chunk_step(j, p % nbuf, p)
            return 0
        lax.fori_loop(0, ngrp2, outer, 0)

        for p in range(tail):
            j = ngrp2 * NI + p
            chunk_step(jnp.int32(j), p % nbuf, p)

        for b in range(nbuf):
            drain_scatter(b)

        plsc.subcore_barrier()

        @pl.when(c == 0)
        def _():
            pltpu.sync_copy(accum.at[pl.ds(rlo, rpt)],
                            z0_hbm.at[pl.ds(rlo, rpt)])

        @pl.when(c == 1)
        def _():
            pltpu.sync_copy(accum.at[pl.ds(rlo, rpt)],
                            z1_hbm.at[pl.ds(rlo, rpt)])

    return k(y0, y1, ecnk)


def _full(shape):
    nd = len(shape)
    return pl.BlockSpec(shape, lambda i: (0,) * nd)


def _row(block):
    return pl.BlockSpec(block, lambda i: (i,) + (0,) * (len(block) - 1))


def _halves_out(fo):
    h = jax.ShapeDtypeStruct((NPAD, fo // 2), jnp.float32)
    return dict(
        out_shape=(h, h),
        out_specs=(_row((R, fo // 2)), _row((R, fo // 2))),
    )


def _tc_layer1(x, W, deg2):
    fo = W.shape[1]

    def body(x_ref, w_ref, d_ref, o0_ref, o1_ref):
        dinv = lax.rsqrt(d_ref[...] + 1.0)
        y = jnp.dot(x_ref[...], w_ref[...],
                    preferred_element_type=jnp.float32) * dinv
        o0_ref[...] = y[:, : fo // 2]
        o1_ref[...] = y[:, fo // 2:]

    return pl.pallas_call(
        body,
        grid=(N // R,),
        in_specs=[_row((R, x.shape[1])), _full(W.shape), _row((R, 1))],
        **_halves_out(fo),
    )(x, W, deg2)


def _tc_layer(z0, z1, b, W, deg2):
    fo = W.shape[1]
    fh = z0.shape[1]

    def body(z0_ref, z1_ref, b_ref, w_ref, d_ref, o0_ref, o1_ref):
        dinv = lax.rsqrt(d_ref[...] + 1.0)
        zc = jnp.concatenate([z0_ref[...], z1_ref[...]], axis=1)
        h = jnp.maximum(zc * dinv + b_ref[...], 0.0)
        y = jnp.dot(h, w_ref[...], preferred_element_type=jnp.float32) * dinv
        o0_ref[...] = y[:, : fo // 2]
        o1_ref[...] = y[:, fo // 2:]

    return pl.pallas_call(
        body,
        grid=(N // R,),
        in_specs=[_row((R, fh)), _row((R, fh)), _full(b.shape),
                  _full(W.shape), _row((R, 1))],
        **_halves_out(fo),
    )(z0, z1, b, W, deg2)


def _tc_final(z0, z1, b, deg2):
    fh = z0.shape[1]

    def body(z0_ref, z1_ref, b_ref, d_ref, o_ref):
        dinv = lax.rsqrt(d_ref[...] + 1.0)
        zc = jnp.concatenate([z0_ref[...], z1_ref[...]], axis=1)
        o_ref[...] = jax.nn.sigmoid(zc * dinv + b_ref[...])

    return pl.pallas_call(
        body,
        grid=(N // R,),
        in_specs=[_row((R, fh)), _row((R, fh)), _full(b.shape), _row((R, 1))],
        out_shape=jax.ShapeDtypeStruct((N, 2 * fh), jnp.float32),
        out_specs=_row((R, 2 * fh)),
    )(z0, z1, b, deg2)


def _chunk_edges(src, dst, ch):
    ept = E // NS
    cpt = -(-ept // ch)
    npe = cpt * ch - ept
    s16 = src.reshape(NS, ept)
    d16 = dst.reshape(NS, ept)
    if npe:
        flat = jnp.arange(NS * npe, dtype=jnp.int32).reshape(NS, npe)
        s16 = jnp.concatenate([s16, flat % N], axis=1)
        d16 = jnp.concatenate([d16, N + flat % (NPAD - N)], axis=1)
    s16 = s16.reshape(NS, cpt, ch)
    d16 = d16.reshape(NS, cpt, ch)
    return jnp.stack([s16, d16], axis=2).reshape(NS * cpt, 2, ch)


def kernel(x, edge_index, W1, b1, W2, b2, W3, b3):
    src = edge_index[0].astype(jnp.int32)
    dst = edge_index[1].astype(jnp.int32)
    ecnk = _chunk_edges(src, dst, CH)
    ecnk3 = _chunk_edges(src, dst, CH3)

    degf = _sc_degree(dst)
    deg2 = (degf[:N] + degf[NPAD:NPAD + N]).reshape(N, 1)

    y0, y1 = _tc_layer1(x, W1, deg2)
    z0, z1 = _sc_propagate(y0, y1, ecnk, 128, CH, NBUF, G)

    y0, y1 = _tc_layer(z0, z1, b1.reshape(1, -1), W2, deg2)
    z0, z1 = _sc_propagate(y0, y1, ecnk, 128, CH, NBUF, G)

    y0, y1 = _tc_layer(z0, z1, b2.reshape(1, -1), W3, deg2)
    z0, z1 = _sc_propagate(y0, y1, ecnk3, 32, CH3, NBUF3, G3)

    return _tc_final(z0, z1, b3.reshape(1, -1), deg2)


# speedup vs baseline: 1.0703x; 1.0115x over previous
import functools

import jax
import jax.numpy as jnp
from jax import lax
from jax.experimental import pallas as pl
from jax.experimental.pallas import tpu as pltpu
import jax.experimental.pallas.tpu_sc as plsc

N = 10000
E = 320000
NC, NS, L = 2, 16, 16
NW = NC * NS
NPAD = 10240
R = 400
CH = 96
NBUF = 3
G = 2
CH3 = 128
NBUF3 = 4
G3 = 2


def _sc_degree(dst):
    epw = E // NW
    cpt = NPAD // NS
    mesh = plsc.VectorSubcoreMesh(core_axis_name="c", subcore_axis_name="s")

    @functools.partial(
        pl.kernel,
        out_type=jax.ShapeDtypeStruct((2 * NPAD,), jnp.float32),
        mesh=mesh,
        scratch_types=[
            pltpu.VMEM((epw,), jnp.int32),
            pltpu.VMEM((NPAD,), jnp.float32),
            pltpu.VMEM((NS * cpt,), jnp.float32),
            pltpu.VMEM((cpt,), jnp.float32),
            pltpu.VMEM_SHARED((NS * NPAD,), jnp.float32),
        ],
        compiler_params=pltpu.CompilerParams(needs_layout_passes=False),
    )
    def k(dst_hbm, out_hbm, dst_v, hist_v, red_v, res_v, shared):
        c = lax.axis_index("c")
        s = lax.axis_index("s")
        wid = s * NC + c

        pltpu.sync_copy(dst_hbm.at[pl.ds(wid * epw, epw)], dst_v)

        def zero_body(j, _):
            hist_v[pl.ds(j * L, L)] = jnp.zeros((L,), jnp.float32)
            return 0
        lax.fori_loop(0, NPAD // L, zero_body, 0)

        ones = jnp.full((L,), 1.0, jnp.float32)

        def count_body(j, _):
            idx = dst_v[pl.ds(j * L, L)]
            plsc.addupdate_scatter(hist_v, [idx], ones)
            return 0
        lax.fori_loop(0, epw // L, count_body, 0)

        pltpu.sync_copy(hist_v, shared.at[pl.ds(s * NPAD, NPAD)])
        plsc.subcore_barrier()

        for t in range(NS):
            pltpu.sync_copy(shared.at[pl.ds(t * NPAD + s * cpt, cpt)],
                            red_v.at[pl.ds(t * cpt, cpt)])

        def red_body(j, _):
            acc = jnp.zeros((L,), jnp.float32)

            def add_t(t, a):
                return a + red_v[pl.ds(t * cpt + j * L, L)]
            acc = lax.fori_loop(0, NS, add_t, acc)
            res_v[pl.ds(j * L, L)] = acc
            return 0
        lax.fori_loop(0, cpt // L, red_body, 0)

        pltpu.sync_copy(res_v, out_hbm.at[pl.ds(c * NPAD + s * cpt, cpt)])

    return k(dst)


def _sc_propagate(y0, y1, ecnk, fh, ch, nbuf, g, stage=False):
    nch = ecnk.shape[0] // NS
    NI = 2 * nbuf
    ngrp2 = nch // NI
    tail = nch - ngrp2 * NI
    rpt = NPAD // NS
    mesh = plsc.VectorSubcoreMesh(core_axis_name="c", subcore_axis_name="s")
    half = jax.ShapeDtypeStruct((NPAD, fh), jnp.float32)

    @functools.partial(
        pl.kernel,
        out_type=(half, half),
        mesh=mesh,
        scratch_types=[
            pltpu.VMEM((NI, 2, ch), jnp.int32),
            pltpu.VMEM((nbuf, ch, fh), jnp.float32),
            pltpu.VMEM_SHARED((NPAD, fh), jnp.float32),
            pltpu.VMEM_SHARED((NPAD if stage else 8, fh), jnp.float32),
            [pltpu.SemaphoreType.DMA] * NI,
            [pltpu.SemaphoreType.DMA] * nbuf,
            [pltpu.SemaphoreType.DMA] * nbuf,
        ],
        compiler_params=pltpu.CompilerParams(needs_layout_passes=False,
                                             use_tc_tiling_on_sc=False),
    )
    def k(y0_hbm, y1_hbm, e_hbm, z0_hbm, z1_hbm,
          ibuf, gbuf, accum, ysp, isems, gsems, ssems):
        c = lax.axis_index("c")
        s = lax.axis_index("s")
        base = s * nch

        rlo = s * rpt

        @pl.when(c == 0)
        def _():
            pltpu.sync_copy(y0_hbm.at[pl.ds(rlo, rpt)],
                            accum.at[pl.ds(rlo, rpt)])
            if stage:
                pltpu.sync_copy(y0_hbm.at[pl.ds(rlo, rpt)],
                                ysp.at[pl.ds(rlo, rpt)])

        @pl.when(c == 1)
        def _():
            pltpu.sync_copy(y1_hbm.at[pl.ds(rlo, rpt)],
                            accum.at[pl.ds(rlo, rpt)])
            if stage:
                pltpu.sync_copy(y1_hbm.at[pl.ds(rlo, rpt)],
                                ysp.at[pl.ds(rlo, rpt)])

        def fire_idx(j, islot):
            pltpu.async_copy(e_hbm.at[base + j], ibuf.at[islot], isems[islot])

        def wait_idx(islot):
            pltpu.make_async_copy(e_hbm.at[0], ibuf.at[islot],
                                  isems[islot]).wait()

        def fire_gather(islot, slot):
            idx = ibuf.at[islot, 0]
            if stage:
                pltpu.async_copy(ysp.at[idx], gbuf.at[slot], gsems[slot])
            else:
                @pl.when(c == 0)
                def _():
                    pltpu.async_copy(y0_hbm.at[idx], gbuf.at[slot],
                                     gsems[slot])

                @pl.when(c == 1)
                def _():
                    pltpu.async_copy(y1_hbm.at[idx], gbuf.at[slot],
                                     gsems[slot])

        def wait_gather(slot):
            src = ysp if stage else y0_hbm
            pltpu.make_async_copy(src.at[pl.ds(0, ch)], gbuf.at[slot],
                                  gsems[slot]).wait()

        def fire_scatter(islot, slot):
            pltpu.async_copy(gbuf.at[slot], accum.at[ibuf.at[islot, 1]],
                             ssems[slot], add=True)

        def drain_scatter(slot):
            pltpu.make_async_copy(gbuf.at[slot], accum.at[ibuf.at[0, 1]],
                                  ssems[slot]).wait()

        plsc.subcore_barrier()

        for b in range(nbuf):
            fire_idx(b, b)
        for b in range(g):
            wait_idx(b)
            fire_gather(b, b)

        def chunk_step(j, b, islot):
            gslot = (b + g) % nbuf
            gislot = (islot + g) % NI

            @pl.when(j < nch - g)
            def _():
                @pl.when(j + g >= nbuf)
                def _():
                    drain_scatter(gslot)
                wait_idx(gislot)
                fire_gather(gislot, gslot)

            wait_gather(b)
            fire_scatter(islot, b)

            @pl.when(j < nch - nbuf)
            def _():
                fire_idx(j + nbuf, (islot + nbuf) % NI)

        def outer(g2, _):
            for p in range(NI):
                j = g2 * NI + p
                chunk_step(j, p % nbuf, p)
            return 0
        lax.fori_loop(0, ngrp2, outer, 0)

        for p in range(tail):
            j = ngrp2 * NI + p
            chunk_step(jnp.int32(j), p % nbuf, p)

        for b in range(nbuf):
            drain_scatter(b)

        plsc.subcore_barrier()

        @pl.when(c == 0)
        def _():
            pltpu.sync_copy(accum.at[pl.ds(rlo, rpt)],
                            z0_hbm.at[pl.ds(rlo, rpt)])

        @pl.when(c == 1)
        def _():
            pltpu.sync_copy(accum.at[pl.ds(rlo, rpt)],
                            z1_hbm.at[pl.ds(rlo, rpt)])

    return k(y0, y1, ecnk)


def _full(shape):
    nd = len(shape)
    return pl.BlockSpec(shape, lambda i: (0,) * nd)


def _row(block):
    return pl.BlockSpec(block, lambda i: (i,) + (0,) * (len(block) - 1))


def _halves_out(fo):
    h = jax.ShapeDtypeStruct((NPAD, fo // 2), jnp.float32)
    return dict(
        out_shape=(h, h),
        out_specs=(_row((R, fo // 2)), _row((R, fo // 2))),
    )


def _tc_layer1(x, W, deg2):
    fo = W.shape[1]

    def body(x_ref, w_ref, d_ref, o0_ref, o1_ref):
        dinv = lax.rsqrt(d_ref[...] + 1.0)
        y = jnp.dot(x_ref[...], w_ref[...],
                    preferred_element_type=jnp.float32) * dinv
        o0_ref[...] = y[:, : fo // 2]
        o1_ref[...] = y[:, fo // 2:]

    return pl.pallas_call(
        body,
        grid=(N // R,),
        in_specs=[_row((R, x.shape[1])), _full(W.shape), _row((R, 1))],
        **_halves_out(fo),
    )(x, W, deg2)


def _tc_layer(z0, z1, b, W, deg2):
    fo = W.shape[1]
    fh = z0.shape[1]

    def body(z0_ref, z1_ref, b_ref, w_ref, d_ref, o0_ref, o1_ref):
        dinv = lax.rsqrt(d_ref[...] + 1.0)
        zc = jnp.concatenate([z0_ref[...], z1_ref[...]], axis=1)
        h = jnp.maximum(zc * dinv + b_ref[...], 0.0)
        y = jnp.dot(h, w_ref[...], preferred_element_type=jnp.float32) * dinv
        o0_ref[...] = y[:, : fo // 2]
        o1_ref[...] = y[:, fo // 2:]

    return pl.pallas_call(
        body,
        grid=(N // R,),
        in_specs=[_row((R, fh)), _row((R, fh)), _full(b.shape),
                  _full(W.shape), _row((R, 1))],
        **_halves_out(fo),
    )(z0, z1, b, W, deg2)


def _tc_final(z0, z1, b, deg2):
    fh = z0.shape[1]

    def body(z0_ref, z1_ref, b_ref, d_ref, o_ref):
        dinv = lax.rsqrt(d_ref[...] + 1.0)
        zc = jnp.concatenate([z0_ref[...], z1_ref[...]], axis=1)
        o_ref[...] = jax.nn.sigmoid(zc * dinv + b_ref[...])

    return pl.pallas_call(
        body,
        grid=(N // R,),
        in_specs=[_row((R, fh)), _row((R, fh)), _full(b.shape), _row((R, 1))],
        out_shape=jax.ShapeDtypeStruct((N, 2 * fh), jnp.float32),
        out_specs=_row((R, 2 * fh)),
    )(z0, z1, b, deg2)


def _chunk_edges(src, dst, ch):
    ept = E // NS
    cpt = -(-ept // ch)
    npe = cpt * ch - ept
    s16 = src.reshape(NS, ept)
    d16 = dst.reshape(NS, ept)
    if npe:
        flat = jnp.arange(NS * npe, dtype=jnp.int32).reshape(NS, npe)
        s16 = jnp.concatenate([s16, flat % N], axis=1)
        d16 = jnp.concatenate([d16, N + flat % (NPAD - N)], axis=1)
    s16 = s16.reshape(NS, cpt, ch)
    d16 = d16.reshape(NS, cpt, ch)
    return jnp.stack([s16, d16], axis=2).reshape(NS * cpt, 2, ch)


def kernel(x, edge_index, W1, b1, W2, b2, W3, b3):
    src = edge_index[0].astype(jnp.int32)
    dst = edge_index[1].astype(jnp.int32)
    ecnk = _chunk_edges(src, dst, CH)
    ecnk3 = _chunk_edges(src, dst, CH3)

    degf = _sc_degree(dst)
    deg2 = (degf[:N] + degf[NPAD:NPAD + N]).reshape(N, 1)

    y0, y1 = _tc_layer1(x, W1, deg2)
    z0, z1 = _sc_propagate(y0, y1, ecnk, 128, CH, NBUF, G)

    y0, y1 = _tc_layer(z0, z1, b1.reshape(1, -1), W2, deg2)
    z0, z1 = _sc_propagate(y0, y1, ecnk, 128, CH, NBUF, G)

    y0, y1 = _tc_layer(z0, z1, b2.reshape(1, -1), W3, deg2)
    z0, z1 = _sc_propagate(y0, y1, ecnk3, 32, CH3, NBUF3, G3, stage=True)

    return _tc_final(z0, z1, b3.reshape(1, -1), deg2)
